# Initial kernel scaffold; baseline (speedup 1.0000x reference)
import jax
import jax.numpy as jnp
from jax.experimental import pallas as pl


def kernel(text_feat, attention_mask, base_tokens, delta_W, delta_b, token_W, token_b):
    raise NotImplementedError("write your pallas kernel here")


# speedup vs baseline: 2.0822x; 2.0822x over previous
import functools

import jax
import jax.numpy as jnp
from jax import lax
from jax.experimental import pallas as pl
from jax.experimental.pallas import tpu as pltpu
from jax.experimental.pallas import tpu_sc as plsc

_B, _L, _D = 32, 2048, 512
_K, _E = 32, 256
_NC, _NS, _LANES = 2, 16, 16


def _sc_select_gather_body(feat_hbm, mask_hbm, gath_hbm, sel_hbm,
                           mask_v, sel_v, gidx_v, rows_v, sem):
    b = lax.axis_index("s") * _NC + lax.axis_index("c")
    pltpu.sync_copy(mask_hbm.at[b], mask_v)

    for c in range(_K // _LANES):
        sel_v[pl.ds(c * _LANES, _LANES)] = jnp.full((_LANES,), _L, jnp.int32)

    def chunk(i, cnt):
        m = mask_v[pl.ds(i * _LANES, _LANES)]
        vmask = m > 0
        ones = vmask.astype(jnp.int32)
        rank = plsc.cumsum(ones) + cnt
        slot = rank - 1
        pos = lax.iota(jnp.int32, _LANES) + i * _LANES
        plsc.store_scatter(sel_v, [slot], pos, mask=vmask & (slot < _K))
        return cnt + jnp.sum(ones)

    lax.fori_loop(0, _L // _LANES, chunk, jnp.int32(0))

    for c in range(_K // _LANES):
        s = sel_v[pl.ds(c * _LANES, _LANES)]
        gidx_v[pl.ds(c * _LANES, _LANES)] = jnp.minimum(s, _L - 1) + b * _L

    pltpu.async_copy(feat_hbm.at[gidx_v], rows_v, sem).wait()
    pltpu.sync_copy(rows_v, gath_hbm.at[b])
    pltpu.sync_copy(sel_v, sel_hbm.at[b])


@functools.cache
def _sc_select_gather():
    return pl.kernel(
        _sc_select_gather_body,
        mesh=plsc.VectorSubcoreMesh(core_axis_name="c", subcore_axis_name="s"),
        compiler_params=pltpu.CompilerParams(needs_layout_passes=False),
        out_type=[
            jax.ShapeDtypeStruct((_B, _K, _D), jnp.float32),
            jax.ShapeDtypeStruct((_B, _K), jnp.int32),
        ],
        scratch_types=[
            pltpu.VMEM((_L,), jnp.int32),
            pltpu.VMEM((_K,), jnp.int32),
            pltpu.VMEM((_K,), jnp.int32),
            pltpu.VMEM((_K, _D), jnp.float32),
            pltpu.SemaphoreType.DMA,
        ],
    )


_BB, _CL = 8, 512


def _reduce_body(maskf_ref, feat_ref, sum_ref, cnt_ref):
    l = pl.program_id(1)

    @pl.when(l == 0)
    def _():
        sum_ref[...] = jnp.zeros_like(sum_ref)
        cnt_ref[...] = jnp.zeros_like(cnt_ref)

    mf = maskf_ref[...]
    x = feat_ref[...]
    sum_ref[...] += jnp.sum(x * mf[:, :, None], axis=1)
    cnt_ref[...] += jnp.sum(mf, axis=1, keepdims=True)


def _reduce_call(maskf, text_feat):
    return pl.pallas_call(
        _reduce_body,
        grid=(_B // _BB, _L // _CL),
        in_specs=[
            pl.BlockSpec((_BB, _CL), lambda b, l: (b, l)),
            pl.BlockSpec((_BB, _CL, _D), lambda b, l: (b, l, 0)),
        ],
        out_specs=[
            pl.BlockSpec((_BB, _D), lambda b, l: (b, 0)),
            pl.BlockSpec((_BB, 1), lambda b, l: (b, 0)),
        ],
        out_shape=[
            jax.ShapeDtypeStruct((_B, _D), jnp.float32),
            jax.ShapeDtypeStruct((_B, 1), jnp.float32),
        ],
    )(maskf, text_feat)


def _proj_body(ps_ref, cnt_ref, gath_ref, valid_ref, dw_ref, db_ref,
               tw_ref, tb_ref, base_ref, out_ref):
    pooled = ps_ref[...] / jnp.maximum(cnt_ref[...], 1.0)
    g = lax.dot_general(pooled, dw_ref[...], (((1,), (1,)), ((), ())),
                        preferred_element_type=jnp.float32)
    t = lax.dot_general(gath_ref[...], tw_ref[...], (((1,), (1,)), ((), ())),
                        preferred_element_type=jnp.float32)
    t = (t + tb_ref[...]) * valid_ref[...]
    out = (base_ref[...] + db_ref[...].reshape(1, _K, _E)
           + g.reshape(_B, _K, _E) + t.reshape(_B, _K, _E))
    out_ref[...] = out


def _proj_call(psum, cnt, gath2d, valid, delta_W, delta_b2d, token_W,
               token_b2d, base_tokens):
    return pl.pallas_call(
        _proj_body,
        out_shape=jax.ShapeDtypeStruct((_B, _K, _E), jnp.float32),
    )(psum, cnt, gath2d, valid, delta_W, delta_b2d, token_W, token_b2d,
      base_tokens)


def kernel(text_feat, attention_mask, base_tokens, delta_W, delta_b,
           token_W, token_b):
    feat_flat = text_feat.reshape(_B * _L, _D)
    gathered, sel = _sc_select_gather()(feat_flat, attention_mask)
    maskf = (attention_mask > 0).astype(jnp.float32)
    psum, cnt = _reduce_call(maskf, text_feat)
    valid = (sel < _L).astype(jnp.float32).reshape(_B * _K, 1)
    out = _proj_call(psum, cnt, gathered.reshape(_B * _K, _D), valid,
                     delta_W, delta_b.reshape(1, _K * _E), token_W,
                     token_b.reshape(1, _E), base_tokens)
    return out
